# baseline (speedup 1.0000x reference)
import functools

import jax
import jax.numpy as jnp
from jax import lax
from jax.experimental import pallas as pl
from jax.experimental.pallas import tpu as pltpu
from jax.experimental.pallas import tpu_sc as plsc

B = 16384
L = 50
D = 64
VK = 100352
NBLK = 32
BB = B // NBLK
LANES = 16


def _sc_info():
    try:
        info = plsc.get_sparse_core_info()
        return info.num_cores, info.num_subcores
    except Exception:
        return 2, 16


RKA = 2048


def _ka_body(tbl_ref, Wk1T_ref, bk1col_ref, wk2row_ref, out_ref):
    hT = jax.lax.dot_general(Wk1T_ref[:], tbl_ref[:],
                             (((1,), (1,)), ((), ())))
    hT = jnp.maximum(hT + bk1col_ref[:], 0.0)
    ek = wk2row_ref[:] @ hT
    rowid = (jax.lax.broadcasted_iota(jnp.int32, (1, RKA), 1)
             + pl.program_id(0) * RKA)
    ek = ek - (rowid == 0).astype(jnp.float32) * 1e8
    out_ref[:] = ek.reshape(RKA)


def _precompute_k(src_iid, Wk1T, bk1col, wk2row):
    return pl.pallas_call(
        _ka_body,
        grid=(VK // RKA,),
        in_specs=[
            pl.BlockSpec((RKA, D), lambda i: (i, 0)),
            pl.BlockSpec((D, D), lambda i: (0, 0)),
            pl.BlockSpec((D, 1), lambda i: (0, 0)),
            pl.BlockSpec((1, D), lambda i: (0, 0)),
        ],
        out_specs=pl.BlockSpec((RKA,), lambda i: (i,)),
        out_shape=jax.ShapeDtypeStruct((VK,), jnp.float32),
    )(src_iid, Wk1T, bk1col, wk2row)


def _sc_softmax_uv(k_tab, seq_t, x0, x1, src_uid, tgt_iid):
    NC, NS = _sc_info()
    NW = NC * NS
    n_b = B // NW
    CB = 16
    NCH = n_b // CB
    NI = CB * L
    CH = 128
    mesh = plsc.VectorSubcoreMesh(core_axis_name="c", subcore_axis_name="s")

    @functools.partial(
        pl.kernel,
        out_type=[
            jax.ShapeDtypeStruct((B * L,), jnp.float32),
            jax.ShapeDtypeStruct((B,), jnp.float32),
            jax.ShapeDtypeStruct((B, D), jnp.float32),
            jax.ShapeDtypeStruct((B, D), jnp.float32),
            jax.ShapeDtypeStruct((B * L,), jnp.int32),
        ],
        mesh=mesh,
        compiler_params=pltpu.CompilerParams(use_tc_tiling_on_sc=False),
        scratch_types=[
            pltpu.VMEM((2, NI), jnp.int32),
            pltpu.VMEM((2, NI), jnp.float32),
            pltpu.VMEM((n_b * L,), jnp.float32),
            pltpu.VMEM((n_b,), jnp.float32),
            pltpu.VMEM((n_b,), jnp.int32),
            pltpu.VMEM((n_b, D), jnp.float32),
            pltpu.SemaphoreType.DMA,
            pltpu.SemaphoreType.DMA,
            pltpu.SemaphoreType.DMA,
        ],
    )
    def k(k_hbm, seq_hbm, x0_hbm, x1_hbm, uid_hbm, tgt_hbm,
          e_hbm, inv_hbm, u_out, v_out, seq_echo,
          sbuf, tbuf, eall, ivall, uvidx, uvrows, sem, gsem0, gsem1):
        wid = lax.axis_index("s") * NC + lax.axis_index("c")
        gsems = (gsem0, gsem1)

        def fire(ci, slot):
            base = (wid * NCH + ci) * NI
            pltpu.sync_copy(seq_hbm.at[pl.ds(base, NI)], sbuf.at[slot])
            for j in range(7):
                n = 128 if j < 6 else 32
                pltpu.async_copy(
                    k_hbm.at[sbuf.at[slot, pl.ds(j * 128, n)]],
                    tbuf.at[slot, pl.ds(j * 128, n)], gsems[slot])

        def proc(ci, slot):
            for j in range(7):
                n = 128 if j < 6 else 32
                pltpu.make_async_copy(
                    k_hbm.at[sbuf.at[slot, pl.ds(j * 128, n)]],
                    tbuf.at[slot, pl.ds(j * 128, n)], gsems[slot]).wait()
            m = tbuf[slot, pl.ds(0, LANES)]
            for l in range(1, L):
                m = jnp.maximum(m, tbuf[slot, pl.ds(l * LANES, LANES)])
            s = jnp.zeros((LANES,), jnp.float32)
            for l in range(L):
                e = jnp.exp(tbuf[slot, pl.ds(l * LANES, LANES)] - m)
                s = s + e
                eall[pl.ds(ci * NI + l * LANES, LANES)] = e
            ivall[pl.ds(ci * CB, LANES)] = (
                jnp.full((LANES,), 1.0, jnp.float32) / s)

        fire(0, 0)

        def pair(i, carry):
            ci0 = i * 2
            fire(ci0 + 1, 1)
            proc(ci0, 0)

            @pl.when(ci0 + 2 < NCH)
            def _():
                fire(ci0 + 2, 0)
            proc(ci0 + 1, 1)
            return carry
        lax.fori_loop(0, NCH // 2, pair, 0)
        pltpu.sync_copy(eall, e_hbm.at[pl.ds(wid * n_b * L, n_b * L)])
        pltpu.sync_copy(ivall, inv_hbm.at[pl.ds(wid * n_b, n_b)])
        pltpu.sync_copy(seq_hbm.at[pl.ds(wid * n_b * L, n_b * L)],
                        seq_echo.at[pl.ds(wid * n_b * L, n_b * L)])

        def gather_rows(idx_hbm, table_hbm, out_hbm):
            pltpu.sync_copy(idx_hbm.at[pl.ds(wid * n_b, n_b)], uvidx)
            for j in range(n_b // CH):
                pltpu.async_copy(table_hbm.at[uvidx.at[pl.ds(j * CH, CH)]],
                                 uvrows.at[pl.ds(j * CH, CH)], sem)
            for j in range(n_b // CH):
                pltpu.make_async_copy(
                    table_hbm.at[uvidx.at[pl.ds(j * CH, CH)]],
                    uvrows.at[pl.ds(j * CH, CH)], sem).wait()
            pltpu.sync_copy(uvrows, out_hbm.at[pl.ds(wid * n_b, n_b)])

        gather_rows(x0_hbm, uid_hbm, u_out)
        gather_rows(x1_hbm, tgt_hbm, v_out)

    return k(k_tab, seq_t, x0, x1, src_uid, tgt_iid)


def _sc_bag(seq_t, e_lmaj, src_iid):
    NC, NS = _sc_info()
    NW = NC * NS
    n_b = B // NW
    CB = 16
    NCH = n_b // CB
    NI = CB * L
    mesh = plsc.VectorSubcoreMesh(core_axis_name="c", subcore_axis_name="s")

    @functools.partial(
        pl.kernel,
        out_type=jax.ShapeDtypeStruct((B, D), jnp.float32),
        mesh=mesh,
        compiler_params=pltpu.CompilerParams(use_tc_tiling_on_sc=False),
        scratch_types=[
            pltpu.VMEM((2, NI), jnp.int32),
            pltpu.VMEM((2, NI), jnp.float32),
            pltpu.VMEM((2, NI, D), jnp.float32),
            pltpu.VMEM((CB, D), jnp.float32),
            pltpu.SemaphoreType.DMA,
            pltpu.SemaphoreType.DMA,
            pltpu.SemaphoreType.DMA,
            pltpu.SemaphoreType.DMA,
        ],
    )
    def k(seq_hbm, e_hbm, iid_hbm, his_out,
          sbuf, ebuf, rows, hisbuf, gsem0, gsem1, csem0, csem1):
        wid = lax.axis_index("s") * NC + lax.axis_index("c")
        gsems = (gsem0, gsem1)
        csems = (csem0, csem1)

        def stage_a(ci, slot):
            base = (wid * NCH + ci) * NI
            pltpu.async_copy(seq_hbm.at[pl.ds(base, NI)], sbuf.at[slot],
                             csems[slot])
            pltpu.async_copy(e_hbm.at[pl.ds(base, NI)], ebuf.at[slot],
                             csems[slot])

        def stage_b(ci, slot):
            base = (wid * NCH + ci) * NI
            pltpu.make_async_copy(seq_hbm.at[pl.ds(base, NI)], sbuf.at[slot],
                                  csems[slot]).wait()
            pltpu.make_async_copy(e_hbm.at[pl.ds(base, NI)], ebuf.at[slot],
                                  csems[slot]).wait()
            for j in range(7):
                n = 128 if j < 6 else 32
                pltpu.async_copy(
                    iid_hbm.at[sbuf.at[slot, pl.ds(j * 128, n)]],
                    rows.at[slot, pl.ds(j * 128, n)],
                    gsems[slot])

        def stage_c(ci, slot):
            b0 = wid * n_b + ci * CB
            for j in range(7):
                n = 128 if j < 6 else 32
                pltpu.make_async_copy(
                    iid_hbm.at[sbuf.at[slot, pl.ds(j * 128, n)]],
                    rows.at[slot, pl.ds(j * 128, n)],
                    gsems[slot]).wait()

            def one_b(bl, carry):
                acc = [jnp.zeros((LANES,), jnp.float32) for _ in range(4)]
                for l in range(L):
                    ev = ebuf[slot, pl.ds(l * LANES, LANES)]
                    w = ev[jnp.full((LANES,), bl, jnp.int32)]
                    for q in range(4):
                        acc[q] = acc[q] + w * rows[slot, l * LANES + bl,
                                                   pl.ds(q * LANES, LANES)]
                for q in range(4):
                    hisbuf[bl, pl.ds(q * LANES, LANES)] = acc[q]
                return carry
            lax.fori_loop(0, CB, one_b, 0)
            pltpu.sync_copy(hisbuf, his_out.at[pl.ds(b0, CB)])

        stage_a(0, 0)
        stage_b(0, 0)
        stage_a(1, 1)

        def pair(i, carry):
            ci0 = i * 2
            stage_b(ci0 + 1, 1)
            stage_c(ci0, 0)

            @pl.when(ci0 + 2 < NCH)
            def _():
                stage_a(ci0 + 2, 0)
                stage_b(ci0 + 2, 0)
            stage_c(ci0 + 1, 1)

            @pl.when(ci0 + 3 < NCH)
            def _():
                stage_a(ci0 + 3, 1)
            return carry
        lax.fori_loop(0, NCH // 2, pair, 0)

    return k(seq_t, e_lmaj, src_iid)


def _tcb_body(his_ref, inv_ref, u_ref, v_ref, Wd1T_ref, bd1_ref, T2_ref,
              B2T_ref, out_ref):
    his = his_ref[:] * inv_ref[:]
    hisT = his.T
    gT = jnp.maximum(Wd1T_ref[:] @ hisT + bd1_ref[:], 0.0)
    uT = u_ref[:].T
    vT = v_ref[:].T
    P2 = T2_ref[:] @ uT
    v2 = jnp.broadcast_to(vT[None], (D, D, BB)).reshape(D * D, BB)
    s3 = jnp.sum((P2 * v2).reshape(D, D, BB), axis=1)
    ub2T = B2T_ref[:] @ uT
    res = jnp.sum(gT * s3, axis=0) + jnp.sum(ub2T * vT, axis=0)
    out_ref[:] = res[None, None, :]


def kernel(x, src_uid, src_iid, tgt_iid, Wk1, bk1, Wk2, Wd1, bd1, Wd2, bd2):
    seq = x[:, 2:]
    seq_t = seq.reshape(B // LANES, LANES, L).transpose(0, 2, 1).reshape(-1)
    k_tab = _precompute_k(src_iid, Wk1.T, bk1.reshape(D, 1), Wk2.reshape(1, D))

    e_lmaj, inv_sum, u_rows, v_rows, seq_echo = _sc_softmax_uv(
        k_tab, seq_t, x[:, 0], x[:, 1], src_uid, tgt_iid)
    his = _sc_bag(seq_echo, e_lmaj, src_iid)

    Wd1T = Wd1.T
    bd1col = bd1.reshape(D, 1)
    T2 = Wd2.reshape(D, D, D).transpose(0, 2, 1).reshape(D * D, D)
    B2T = bd2.reshape(D, D).T
    inv2d = inv_sum.reshape(B, 1)

    out2d = pl.pallas_call(
        _tcb_body,
        grid=(NBLK,),
        in_specs=[
            pl.BlockSpec((BB, D), lambda i: (i, 0)),
            pl.BlockSpec((BB, 1), lambda i: (i, 0)),
            pl.BlockSpec((BB, D), lambda i: (i, 0)),
            pl.BlockSpec((BB, D), lambda i: (i, 0)),
            pl.BlockSpec((D, D), lambda i: (0, 0)),
            pl.BlockSpec((D, 1), lambda i: (0, 0)),
            pl.BlockSpec((D * D, D), lambda i: (0, 0)),
            pl.BlockSpec((D, D), lambda i: (0, 0)),
        ],
        out_specs=pl.BlockSpec((1, 1, BB), lambda i: (i, 0, 0)),
        out_shape=jax.ShapeDtypeStruct((NBLK, 1, BB), jnp.float32),
    )(his, inv2d, u_rows, v_rows, Wd1T, bd1col, T2, B2T)
    return out2d.reshape(B)


# speedup vs baseline: 1.1740x; 1.1740x over previous
import functools

import jax
import jax.numpy as jnp
from jax import lax
from jax.experimental import pallas as pl
from jax.experimental.pallas import tpu as pltpu
from jax.experimental.pallas import tpu_sc as plsc

B = 16384
L = 50
D = 64
VK = 100352
NBLK = 32
BB = B // NBLK
LANES = 16


def _sc_info():
    try:
        info = plsc.get_sparse_core_info()
        return info.num_cores, info.num_subcores
    except Exception:
        return 2, 16


RKA = 2048


def _ka_body(tbl_ref, Wk1T_ref, bk1col_ref, wk2row_ref, out_ref):
    hT = jax.lax.dot_general(Wk1T_ref[:], tbl_ref[:],
                             (((1,), (1,)), ((), ())))
    hT = jnp.maximum(hT + bk1col_ref[:], 0.0)
    ek = wk2row_ref[:] @ hT
    rowid = (jax.lax.broadcasted_iota(jnp.int32, (1, RKA), 1)
             + pl.program_id(0) * RKA)
    ek = ek - (rowid == 0).astype(jnp.float32) * 1e8
    out_ref[:] = ek.reshape(RKA)


def _precompute_k(src_iid, Wk1T, bk1col, wk2row):
    return pl.pallas_call(
        _ka_body,
        grid=(VK // RKA,),
        in_specs=[
            pl.BlockSpec((RKA, D), lambda i: (i, 0)),
            pl.BlockSpec((D, D), lambda i: (0, 0)),
            pl.BlockSpec((D, 1), lambda i: (0, 0)),
            pl.BlockSpec((1, D), lambda i: (0, 0)),
        ],
        out_specs=pl.BlockSpec((RKA,), lambda i: (i,)),
        out_shape=jax.ShapeDtypeStruct((VK,), jnp.float32),
    )(src_iid, Wk1T, bk1col, wk2row)


def _sc_softmax_uv(k_tab, seq_t, x0, x1, src_uid, tgt_iid):
    NC, NS = _sc_info()
    NW = NC * NS
    n_b = B // NW
    CB = 16
    NCH = n_b // CB
    NI = CB * L
    CH = 128
    mesh = plsc.VectorSubcoreMesh(core_axis_name="c", subcore_axis_name="s")

    @functools.partial(
        pl.kernel,
        out_type=[
            jax.ShapeDtypeStruct((B * L,), jnp.float32),
            jax.ShapeDtypeStruct((B,), jnp.float32),
            jax.ShapeDtypeStruct((B, D), jnp.float32),
            jax.ShapeDtypeStruct((B, D), jnp.float32),
        ],
        mesh=mesh,
        compiler_params=pltpu.CompilerParams(use_tc_tiling_on_sc=False),
        scratch_types=[
            pltpu.VMEM((2, NI), jnp.int32),
            pltpu.VMEM((2, NI), jnp.float32),
            pltpu.VMEM((n_b * L,), jnp.float32),
            pltpu.VMEM((n_b,), jnp.float32),
            pltpu.VMEM((n_b,), jnp.int32),
            pltpu.VMEM((n_b, D), jnp.float32),
            pltpu.SemaphoreType.DMA,
            pltpu.SemaphoreType.DMA,
            pltpu.SemaphoreType.DMA,
        ],
    )
    def k(k_hbm, seq_hbm, x0_hbm, x1_hbm, uid_hbm, tgt_hbm,
          e_hbm, inv_hbm, u_out, v_out,
          sbuf, tbuf, eall, ivall, uvidx, uvrows, sem, gsem0, gsem1):
        wid = lax.axis_index("s") * NC + lax.axis_index("c")
        gsems = (gsem0, gsem1)

        def fire(ci, slot):
            base = (wid * NCH + ci) * NI
            pltpu.sync_copy(seq_hbm.at[pl.ds(base, NI)], sbuf.at[slot])
            for j in range(7):
                n = 128 if j < 6 else 32
                pltpu.async_copy(
                    k_hbm.at[sbuf.at[slot, pl.ds(j * 128, n)]],
                    tbuf.at[slot, pl.ds(j * 128, n)], gsems[slot])

        def proc(ci, slot):
            for j in range(7):
                n = 128 if j < 6 else 32
                pltpu.make_async_copy(
                    k_hbm.at[sbuf.at[slot, pl.ds(j * 128, n)]],
                    tbuf.at[slot, pl.ds(j * 128, n)], gsems[slot]).wait()
            m = tbuf[slot, pl.ds(0, LANES)]
            for l in range(1, L):
                m = jnp.maximum(m, tbuf[slot, pl.ds(l * LANES, LANES)])
            s = jnp.zeros((LANES,), jnp.float32)
            for l in range(L):
                e = jnp.exp(tbuf[slot, pl.ds(l * LANES, LANES)] - m)
                s = s + e
                eall[pl.ds(ci * NI + l * LANES, LANES)] = e
            ivall[pl.ds(ci * CB, LANES)] = (
                jnp.full((LANES,), 1.0, jnp.float32) / s)

        fire(0, 0)

        def pair(i, carry):
            ci0 = i * 2
            fire(ci0 + 1, 1)
            proc(ci0, 0)

            @pl.when(ci0 + 2 < NCH)
            def _():
                fire(ci0 + 2, 0)
            proc(ci0 + 1, 1)
            return carry
        lax.fori_loop(0, NCH // 2, pair, 0)
        pltpu.sync_copy(eall, e_hbm.at[pl.ds(wid * n_b * L, n_b * L)])
        pltpu.sync_copy(ivall, inv_hbm.at[pl.ds(wid * n_b, n_b)])

        def gather_rows(idx_hbm, table_hbm, out_hbm):
            pltpu.sync_copy(idx_hbm.at[pl.ds(wid * n_b, n_b)], uvidx)
            for j in range(n_b // CH):
                pltpu.async_copy(table_hbm.at[uvidx.at[pl.ds(j * CH, CH)]],
                                 uvrows.at[pl.ds(j * CH, CH)], sem)
            for j in range(n_b // CH):
                pltpu.make_async_copy(
                    table_hbm.at[uvidx.at[pl.ds(j * CH, CH)]],
                    uvrows.at[pl.ds(j * CH, CH)], sem).wait()
            pltpu.sync_copy(uvrows, out_hbm.at[pl.ds(wid * n_b, n_b)])

        gather_rows(x0_hbm, uid_hbm, u_out)
        gather_rows(x1_hbm, tgt_hbm, v_out)

    return k(k_tab, seq_t, x0, x1, src_uid, tgt_iid)


def _sc_bag(seq_t, e_lmaj, src_iid):
    NC, NS = _sc_info()
    NW = NC * NS
    n_b = B // NW
    CB = 16
    NCH = n_b // CB
    NI = CB * L
    mesh = plsc.VectorSubcoreMesh(core_axis_name="c", subcore_axis_name="s")

    @functools.partial(
        pl.kernel,
        out_type=jax.ShapeDtypeStruct((B, D), jnp.float32),
        mesh=mesh,
        compiler_params=pltpu.CompilerParams(use_tc_tiling_on_sc=False),
        scratch_types=[
            pltpu.VMEM((2, NI), jnp.int32),
            pltpu.VMEM((2, NI), jnp.float32),
            pltpu.VMEM((2, NI, D), jnp.float32),
            pltpu.VMEM((CB, D), jnp.float32),
            pltpu.SemaphoreType.DMA,
            pltpu.SemaphoreType.DMA,
            pltpu.SemaphoreType.DMA,
            pltpu.SemaphoreType.DMA,
        ],
    )
    def k(seq_hbm, e_hbm, iid_hbm, his_out,
          sbuf, ebuf, rows, hisbuf, gsem0, gsem1, csem0, csem1):
        wid = lax.axis_index("s") * NC + lax.axis_index("c")
        gsems = (gsem0, gsem1)
        csems = (csem0, csem1)

        def stage_a(ci, slot):
            base = (wid * NCH + ci) * NI
            pltpu.async_copy(seq_hbm.at[pl.ds(base, NI)], sbuf.at[slot],
                             csems[slot])
            pltpu.async_copy(e_hbm.at[pl.ds(base, NI)], ebuf.at[slot],
                             csems[slot])

        def stage_b(ci, slot):
            base = (wid * NCH + ci) * NI
            pltpu.make_async_copy(seq_hbm.at[pl.ds(base, NI)], sbuf.at[slot],
                                  csems[slot]).wait()
            pltpu.make_async_copy(e_hbm.at[pl.ds(base, NI)], ebuf.at[slot],
                                  csems[slot]).wait()
            for j in range(7):
                n = 128 if j < 6 else 32
                pltpu.async_copy(
                    iid_hbm.at[sbuf.at[slot, pl.ds(j * 128, n)]],
                    rows.at[slot, pl.ds(j * 128, n)],
                    gsems[slot])

        def stage_c(ci, slot):
            b0 = wid * n_b + ci * CB
            for j in range(7):
                n = 128 if j < 6 else 32
                pltpu.make_async_copy(
                    iid_hbm.at[sbuf.at[slot, pl.ds(j * 128, n)]],
                    rows.at[slot, pl.ds(j * 128, n)],
                    gsems[slot]).wait()

            def one_b(bl, carry):
                acc = [jnp.zeros((LANES,), jnp.float32) for _ in range(4)]
                for l in range(L):
                    ev = ebuf[slot, pl.ds(l * LANES, LANES)]
                    w = ev[jnp.full((LANES,), bl, jnp.int32)]
                    for q in range(4):
                        acc[q] = acc[q] + w * rows[slot, l * LANES + bl,
                                                   pl.ds(q * LANES, LANES)]
                for q in range(4):
                    hisbuf[bl, pl.ds(q * LANES, LANES)] = acc[q]
                return carry
            lax.fori_loop(0, CB, one_b, 0)
            pltpu.sync_copy(hisbuf, his_out.at[pl.ds(b0, CB)])

        stage_a(0, 0)
        stage_b(0, 0)
        stage_a(1, 1)

        def pair(i, carry):
            ci0 = i * 2
            stage_b(ci0 + 1, 1)
            stage_c(ci0, 0)

            @pl.when(ci0 + 2 < NCH)
            def _():
                stage_a(ci0 + 2, 0)
                stage_b(ci0 + 2, 0)
            stage_c(ci0 + 1, 1)

            @pl.when(ci0 + 3 < NCH)
            def _():
                stage_a(ci0 + 3, 1)
            return carry
        lax.fori_loop(0, NCH // 2, pair, 0)

    return k(seq_t, e_lmaj, src_iid)


def _tcb_body(his_ref, inv_ref, u_ref, v_ref, Wd1T_ref, bd1_ref, T2_ref,
              B2T_ref, out_ref):
    his = his_ref[:] * inv_ref[:]
    hisT = his.T
    gT = jnp.maximum(Wd1T_ref[:] @ hisT + bd1_ref[:], 0.0)
    uT = u_ref[:].T
    vT = v_ref[:].T
    P2 = T2_ref[:] @ uT
    v2 = jnp.broadcast_to(vT[None], (D, D, BB)).reshape(D * D, BB)
    s3 = jnp.sum((P2 * v2).reshape(D, D, BB), axis=1)
    ub2T = B2T_ref[:] @ uT
    res = jnp.sum(gT * s3, axis=0) + jnp.sum(ub2T * vT, axis=0)
    out_ref[:] = res[None, None, :]


def kernel(x, src_uid, src_iid, tgt_iid, Wk1, bk1, Wk2, Wd1, bd1, Wd2, bd2):
    seq = x[:, 2:]
    seq_t = seq.reshape(B // LANES, LANES, L).transpose(0, 2, 1).reshape(-1)
    k_tab = _precompute_k(src_iid, Wk1.T, bk1.reshape(D, 1), Wk2.reshape(1, D))

    e_lmaj, inv_sum, u_rows, v_rows = _sc_softmax_uv(
        k_tab, seq_t, x[:, 0], x[:, 1], src_uid, tgt_iid)
    his = _sc_bag(seq_t, e_lmaj, src_iid)

    Wd1T = Wd1.T
    bd1col = bd1.reshape(D, 1)
    T2 = Wd2.reshape(D, D, D).transpose(0, 2, 1).reshape(D * D, D)
    B2T = bd2.reshape(D, D).T
    inv2d = inv_sum.reshape(B, 1)

    out2d = pl.pallas_call(
        _tcb_body,
        grid=(NBLK,),
        in_specs=[
            pl.BlockSpec((BB, D), lambda i: (i, 0)),
            pl.BlockSpec((BB, 1), lambda i: (i, 0)),
            pl.BlockSpec((BB, D), lambda i: (i, 0)),
            pl.BlockSpec((BB, D), lambda i: (i, 0)),
            pl.BlockSpec((D, D), lambda i: (0, 0)),
            pl.BlockSpec((D, 1), lambda i: (0, 0)),
            pl.BlockSpec((D * D, D), lambda i: (0, 0)),
            pl.BlockSpec((D, D), lambda i: (0, 0)),
        ],
        out_specs=pl.BlockSpec((1, 1, BB), lambda i: (i, 0, 0)),
        out_shape=jax.ShapeDtypeStruct((NBLK, 1, BB), jnp.float32),
    )(his, inv2d, u_rows, v_rows, Wd1T, bd1col, T2, B2T)
    return out2d.reshape(B)


# speedup vs baseline: 1.2403x; 1.0565x over previous
import functools

import jax
import jax.numpy as jnp
from jax import lax
from jax.experimental import pallas as pl
from jax.experimental.pallas import tpu as pltpu
from jax.experimental.pallas import tpu_sc as plsc

B = 16384
L = 50
D = 64
VK = 100352
NBLK = 32
BB = B // NBLK
LANES = 16


def _sc_info():
    try:
        info = plsc.get_sparse_core_info()
        return info.num_cores, info.num_subcores
    except Exception:
        return 2, 16


RKA = 2048


def _ka_body(tbl_ref, Wk1T_ref, bk1col_ref, wk2row_ref, out_ref):
    hT = jax.lax.dot_general(Wk1T_ref[:], tbl_ref[:],
                             (((1,), (1,)), ((), ())))
    hT = jnp.maximum(hT + bk1col_ref[:], 0.0)
    ek = wk2row_ref[:] @ hT
    rowid = (jax.lax.broadcasted_iota(jnp.int32, (1, RKA), 1)
             + pl.program_id(0) * RKA)
    ek = ek - (rowid == 0).astype(jnp.float32) * 1e8
    out_ref[:] = ek.reshape(RKA)


def _precompute_k(src_iid, Wk1T, bk1col, wk2row):
    return pl.pallas_call(
        _ka_body,
        grid=(VK // RKA,),
        in_specs=[
            pl.BlockSpec((RKA, D), lambda i: (i, 0)),
            pl.BlockSpec((D, D), lambda i: (0, 0)),
            pl.BlockSpec((D, 1), lambda i: (0, 0)),
            pl.BlockSpec((1, D), lambda i: (0, 0)),
        ],
        out_specs=pl.BlockSpec((RKA,), lambda i: (i,)),
        out_shape=jax.ShapeDtypeStruct((VK,), jnp.float32),
    )(src_iid, Wk1T, bk1col, wk2row)


def _sc_softmax_uv(k_tab, seq_t, x0, x1, src_uid, tgt_iid):
    NC, NS = _sc_info()
    NW = NC * NS
    n_b = B // NW
    CB = 16
    NCH = n_b // CB
    NI = CB * L
    CH = 128
    mesh = plsc.VectorSubcoreMesh(core_axis_name="c", subcore_axis_name="s")

    @functools.partial(
        pl.kernel,
        out_type=[
            jax.ShapeDtypeStruct((B * L,), jnp.float32),
            jax.ShapeDtypeStruct((B,), jnp.float32),
            jax.ShapeDtypeStruct((B, D), jnp.float32),
            jax.ShapeDtypeStruct((B, D), jnp.float32),
        ],
        mesh=mesh,
        compiler_params=pltpu.CompilerParams(use_tc_tiling_on_sc=False),
        scratch_types=[
            pltpu.VMEM((2, NI), jnp.int32),
            pltpu.VMEM((2, NI), jnp.float32),
            pltpu.VMEM((n_b * L,), jnp.float32),
            pltpu.VMEM((n_b,), jnp.float32),
            pltpu.VMEM((n_b,), jnp.int32),
            pltpu.VMEM((n_b, D), jnp.float32),
            pltpu.SemaphoreType.DMA,
            pltpu.SemaphoreType.DMA,
            pltpu.SemaphoreType.DMA,
        ],
    )
    def k(k_hbm, seq_hbm, x0_hbm, x1_hbm, uid_hbm, tgt_hbm,
          e_hbm, inv_hbm, u_out, v_out,
          sbuf, tbuf, eall, ivall, uvidx, uvrows, sem, gsem0, gsem1):
        wid = lax.axis_index("s") * NC + lax.axis_index("c")
        gsems = (gsem0, gsem1)

        def fire(ci, slot):
            base = (wid * NCH + ci) * NI
            pltpu.sync_copy(seq_hbm.at[pl.ds(base, NI)], sbuf.at[slot])
            for j in range(7):
                n = 128 if j < 6 else 32
                pltpu.async_copy(
                    k_hbm.at[sbuf.at[slot, pl.ds(j * 128, n)]],
                    tbuf.at[slot, pl.ds(j * 128, n)], gsems[slot])

        def proc(ci, slot):
            for j in range(7):
                n = 128 if j < 6 else 32
                pltpu.make_async_copy(
                    k_hbm.at[sbuf.at[slot, pl.ds(j * 128, n)]],
                    tbuf.at[slot, pl.ds(j * 128, n)], gsems[slot]).wait()
            m = tbuf[slot, pl.ds(0, LANES)]
            for l in range(1, L):
                m = jnp.maximum(m, tbuf[slot, pl.ds(l * LANES, LANES)])
            s = jnp.zeros((LANES,), jnp.float32)
            for l in range(L):
                e = jnp.exp(tbuf[slot, pl.ds(l * LANES, LANES)] - m)
                s = s + e
                eall[pl.ds(ci * NI + l * LANES, LANES)] = e
            ivall[pl.ds(ci * CB, LANES)] = (
                jnp.full((LANES,), 1.0, jnp.float32) / s)

        fire(0, 0)

        def pair(i, carry):
            ci0 = i * 2
            fire(ci0 + 1, 1)
            proc(ci0, 0)

            @pl.when(ci0 + 2 < NCH)
            def _():
                fire(ci0 + 2, 0)
            proc(ci0 + 1, 1)
            return carry
        lax.fori_loop(0, NCH // 2, pair, 0)
        pltpu.sync_copy(eall, e_hbm.at[pl.ds(wid * n_b * L, n_b * L)])
        pltpu.sync_copy(ivall, inv_hbm.at[pl.ds(wid * n_b, n_b)])

        def gather_rows(idx_hbm, table_hbm, out_hbm):
            pltpu.sync_copy(idx_hbm.at[pl.ds(wid * n_b, n_b)], uvidx)
            for j in range(n_b // CH):
                pltpu.async_copy(table_hbm.at[uvidx.at[pl.ds(j * CH, CH)]],
                                 uvrows.at[pl.ds(j * CH, CH)], sem)
            for j in range(n_b // CH):
                pltpu.make_async_copy(
                    table_hbm.at[uvidx.at[pl.ds(j * CH, CH)]],
                    uvrows.at[pl.ds(j * CH, CH)], sem).wait()
            pltpu.sync_copy(uvrows, out_hbm.at[pl.ds(wid * n_b, n_b)])

        gather_rows(x0_hbm, uid_hbm, u_out)
        gather_rows(x1_hbm, tgt_hbm, v_out)

    return k(k_tab, seq_t, x0, x1, src_uid, tgt_iid)


def _sc_bag(seq_t, e_lmaj, src_iid):
    NC, NS = _sc_info()
    NW = NC * NS
    n_b = B // NW
    CB = 16
    NCH = n_b // CB
    NI = CB * L
    mesh = plsc.VectorSubcoreMesh(core_axis_name="c", subcore_axis_name="s")

    @functools.partial(
        pl.kernel,
        out_type=jax.ShapeDtypeStruct((B, D), jnp.float32),
        mesh=mesh,
        compiler_params=pltpu.CompilerParams(use_tc_tiling_on_sc=False),
        scratch_types=[
            pltpu.VMEM((2, NI), jnp.int32),
            pltpu.VMEM((2, NI), jnp.float32),
            pltpu.VMEM((2, NI, D), jnp.float32),
            pltpu.VMEM((CB, D), jnp.float32),
            pltpu.SemaphoreType.DMA,
            pltpu.SemaphoreType.DMA,
            pltpu.SemaphoreType.DMA,
            pltpu.SemaphoreType.DMA,
        ],
    )
    def k(seq_hbm, e_hbm, iid_hbm, his_out,
          sbuf, ebuf, rows, hisbuf, gsem0, gsem1, csem0, csem1):
        wid = lax.axis_index("s") * NC + lax.axis_index("c")
        gsems = (gsem0, gsem1)
        csems = (csem0, csem1)

        def stage_a(ci, slot):
            base = (wid * NCH + ci) * NI
            pltpu.async_copy(seq_hbm.at[pl.ds(base, NI)], sbuf.at[slot],
                             csems[slot])
            pltpu.async_copy(e_hbm.at[pl.ds(base, NI)], ebuf.at[slot],
                             csems[slot])

        def stage_b(ci, slot):
            base = (wid * NCH + ci) * NI
            pltpu.make_async_copy(seq_hbm.at[pl.ds(base, NI)], sbuf.at[slot],
                                  csems[slot]).wait()
            pltpu.make_async_copy(e_hbm.at[pl.ds(base, NI)], ebuf.at[slot],
                                  csems[slot]).wait()
            for j in range(7):
                n = 128 if j < 6 else 32
                pltpu.async_copy(
                    iid_hbm.at[sbuf.at[slot, pl.ds(j * 128, n)]],
                    rows.at[slot, pl.ds(j * 128, n)],
                    gsems[slot])

        def stage_c(ci, slot):
            b0 = wid * n_b + ci * CB
            for j in range(7):
                n = 128 if j < 6 else 32
                pltpu.make_async_copy(
                    iid_hbm.at[sbuf.at[slot, pl.ds(j * 128, n)]],
                    rows.at[slot, pl.ds(j * 128, n)],
                    gsems[slot]).wait()

            def one_pair(bp, carry):
                b0l = bp * 2
                b1l = bp * 2 + 1
                acc0 = [jnp.zeros((LANES,), jnp.float32) for _ in range(4)]
                acc1 = [jnp.zeros((LANES,), jnp.float32) for _ in range(4)]
                for l in range(L):
                    ev = ebuf[slot, pl.ds(l * LANES, LANES)]
                    w0 = ev[jnp.full((LANES,), b0l, jnp.int32)]
                    w1 = ev[jnp.full((LANES,), b1l, jnp.int32)]
                    for q in range(4):
                        acc0[q] = acc0[q] + w0 * rows[slot, l * LANES + b0l,
                                                      pl.ds(q * LANES, LANES)]
                        acc1[q] = acc1[q] + w1 * rows[slot, l * LANES + b1l,
                                                      pl.ds(q * LANES, LANES)]
                for q in range(4):
                    hisbuf[b0l, pl.ds(q * LANES, LANES)] = acc0[q]
                    hisbuf[b1l, pl.ds(q * LANES, LANES)] = acc1[q]
                return carry
            lax.fori_loop(0, CB // 2, one_pair, 0)
            pltpu.sync_copy(hisbuf, his_out.at[pl.ds(b0, CB)])

        stage_a(0, 0)
        stage_b(0, 0)
        stage_a(1, 1)

        def pair(i, carry):
            ci0 = i * 2
            stage_b(ci0 + 1, 1)
            stage_c(ci0, 0)

            @pl.when(ci0 + 2 < NCH)
            def _():
                stage_a(ci0 + 2, 0)
                stage_b(ci0 + 2, 0)
            stage_c(ci0 + 1, 1)

            @pl.when(ci0 + 3 < NCH)
            def _():
                stage_a(ci0 + 3, 1)
            return carry
        lax.fori_loop(0, NCH // 2, pair, 0)

    return k(seq_t, e_lmaj, src_iid)


def _tcb_body(his_ref, inv_ref, u_ref, v_ref, Wd1T_ref, bd1_ref, T2_ref,
              B2T_ref, out_ref):
    his = his_ref[:] * inv_ref[:]
    hisT = his.T
    gT = jnp.maximum(Wd1T_ref[:] @ hisT + bd1_ref[:], 0.0)
    uT = u_ref[:].T
    vT = v_ref[:].T
    P2 = T2_ref[:] @ uT
    v2 = jnp.broadcast_to(vT[None], (D, D, BB)).reshape(D * D, BB)
    s3 = jnp.sum((P2 * v2).reshape(D, D, BB), axis=1)
    ub2T = B2T_ref[:] @ uT
    res = jnp.sum(gT * s3, axis=0) + jnp.sum(ub2T * vT, axis=0)
    out_ref[:] = res[None, None, :]


def kernel(x, src_uid, src_iid, tgt_iid, Wk1, bk1, Wk2, Wd1, bd1, Wd2, bd2):
    seq = x[:, 2:]
    seq_t = seq.reshape(B // LANES, LANES, L).transpose(0, 2, 1).reshape(-1)
    k_tab = _precompute_k(src_iid, Wk1.T, bk1.reshape(D, 1), Wk2.reshape(1, D))

    e_lmaj, inv_sum, u_rows, v_rows = _sc_softmax_uv(
        k_tab, seq_t, x[:, 0], x[:, 1], src_uid, tgt_iid)
    his = _sc_bag(seq_t, e_lmaj, src_iid)

    Wd1T = Wd1.T
    bd1col = bd1.reshape(D, 1)
    T2 = Wd2.reshape(D, D, D).transpose(0, 2, 1).reshape(D * D, D)
    B2T = bd2.reshape(D, D).T
    inv2d = inv_sum.reshape(B, 1)

    out2d = pl.pallas_call(
        _tcb_body,
        grid=(NBLK,),
        in_specs=[
            pl.BlockSpec((BB, D), lambda i: (i, 0)),
            pl.BlockSpec((BB, 1), lambda i: (i, 0)),
            pl.BlockSpec((BB, D), lambda i: (i, 0)),
            pl.BlockSpec((BB, D), lambda i: (i, 0)),
            pl.BlockSpec((D, D), lambda i: (0, 0)),
            pl.BlockSpec((D, 1), lambda i: (0, 0)),
            pl.BlockSpec((D * D, D), lambda i: (0, 0)),
            pl.BlockSpec((D, D), lambda i: (0, 0)),
        ],
        out_specs=pl.BlockSpec((1, 1, BB), lambda i: (i, 0, 0)),
        out_shape=jax.ShapeDtypeStruct((NBLK, 1, BB), jnp.float32),
    )(his, inv2d, u_rows, v_rows, Wd1T, bd1col, T2, B2T)
    return out2d.reshape(B)
